# baseline (speedup 1.0000x reference)
import functools

import jax
import jax.numpy as jnp
from jax import lax
from jax.experimental import pallas as pl
from jax.experimental.pallas import tpu as pltpu
from jax.experimental.pallas import tpu_sc as plsc

B = 16384
D = 32

_info = plsc.get_sparse_core_info()
NC, NS, L = _info.num_cores, _info.num_subcores, _info.num_lanes
NW = NC * NS
BPW = B // NW
CH = 128
NCH = BPW // CH


def _sc_scores_kernel(users_hbm, pos_hbm, neg_hbm, utab_hbm, itab_hbm,
                      out_hbm, idx_u, idx_p, idx_n,
                      rows_u, rows_p, rows_n, scores_v, sem):
    wid = lax.axis_index("s") * NC + lax.axis_index("c")
    base = wid * BPW

    for c in range(NCH):
        off = base + c * CH
        pltpu.sync_copy(users_hbm.at[pl.ds(off, CH)], idx_u.at[c])
        pltpu.sync_copy(pos_hbm.at[pl.ds(off, CH)], idx_p.at[c])
        pltpu.sync_copy(neg_hbm.at[pl.ds(off, CH)], idx_n.at[c])

    copies = []
    for c in range(NCH):
        r = pl.ds(c * CH, CH)
        copies.append(pltpu.async_copy(utab_hbm.at[idx_u.at[c]], rows_u.at[r], sem))
        copies.append(pltpu.async_copy(itab_hbm.at[idx_p.at[c]], rows_p.at[r], sem))
        copies.append(pltpu.async_copy(itab_hbm.at[idx_n.at[c]], rows_n.at[r], sem))
    for cp in copies:
        cp.wait()

    def body(g, carry):
        row_ids = g * L + lax.iota(jnp.int32, L)
        acc = jnp.zeros((L,), jnp.float32)
        for j in range(D):
            col = jnp.full((L,), j, jnp.int32)
            u = plsc.load_gather(rows_u, [row_ids, col])
            p = plsc.load_gather(rows_p, [row_ids, col])
            n = plsc.load_gather(rows_n, [row_ids, col])
            acc = acc + u * (n - p)
        scores_v[pl.ds(g * L, L)] = acc
        return carry

    lax.fori_loop(0, BPW // L, body, 0)
    pltpu.sync_copy(scores_v, out_hbm.at[pl.ds(base, BPW)])


_sc_scores = functools.partial(
    pl.kernel,
    mesh=plsc.VectorSubcoreMesh(core_axis_name="c", subcore_axis_name="s"),
    out_type=jax.ShapeDtypeStruct((B,), jnp.float32),
    scratch_types=[
        pltpu.VMEM((NCH, CH), jnp.int32),
        pltpu.VMEM((NCH, CH), jnp.int32),
        pltpu.VMEM((NCH, CH), jnp.int32),
        pltpu.VMEM((BPW, D), jnp.float32),
        pltpu.VMEM((BPW, D), jnp.float32),
        pltpu.VMEM((BPW, D), jnp.float32),
        pltpu.VMEM((BPW,), jnp.float32),
        pltpu.SemaphoreType.DMA,
    ],
    compiler_params=pltpu.CompilerParams(
        needs_layout_passes=False, use_tc_tiling_on_sc=False),
)(_sc_scores_kernel)


def _softplus_mean_kernel(x_ref, o_ref):
    x = x_ref[...]
    o_ref[...] = (jnp.sum(jnp.log(1.0 + jnp.exp(x))) * (1.0 / B))[None, None]


def kernel(users, positive_items, negative_items, user_embedding, item_embedding):
    scores = _sc_scores(users, positive_items, negative_items,
                        user_embedding, item_embedding)
    loss = pl.pallas_call(
        _softplus_mean_kernel,
        out_shape=jax.ShapeDtypeStruct((1, 1), jnp.float32),
    )(scores.reshape(128, 128))
    return loss.reshape(())
